# baseline (speedup 1.0000x reference)
import jax
import jax.numpy as jnp
from jax.experimental import pallas as pl
from jax.experimental.pallas import tpu as pltpu

_BM = 400
_BK = 4096


def kernel(x, adj, W1, b1, W2):
    n, nfeat = x.shape
    nhid = W1.shape[1]
    nclass = W2.shape[1]
    bm = _BM
    bk = _BK
    nblk = n // bm
    nkblk = -(-n // bk)
    n_pad = nkblk * bk
    ncat = nhid + nclass

    def _pass_a_kernel(adj_ref, x_ref, w1_ref, b1_ref, w2_ref,
                       hid_ref, s2_ref, part_ref, cat_ref):
        t = pl.program_id(0)

        @pl.when(t == 0)
        def _():
            cat_ref[:, nhid:] = jnp.zeros((n, nclass), jnp.float32)
            cat_ref[:, :nhid] = jnp.dot(x_ref[...], w1_ref[...],
                                        preferred_element_type=jnp.float32)

        both = jnp.dot(adj_ref[...], cat_ref[...],
                       preferred_element_type=jnp.float32)
        h = jnp.maximum(both[:, :nhid] + b1_ref[...], 0.0)
        hid_ref[...] = h
        part_ref[...] = both[:, nhid:]
        s2_blk = jnp.dot(h, w2_ref[...], preferred_element_type=jnp.float32)
        cat_ref[pl.ds(t * bm, bm), nhid:] = s2_blk
        s2_ref[...] = s2_blk

    hid, s2, part = pl.pallas_call(
        _pass_a_kernel,
        grid=(nblk,),
        in_specs=[pl.BlockSpec((bm, n), lambda t: (t, 0)),
                  pl.BlockSpec((n, nfeat), lambda t: (0, 0)),
                  pl.BlockSpec((nfeat, nhid), lambda t: (0, 0)),
                  pl.BlockSpec((1, nhid), lambda t: (0, 0)),
                  pl.BlockSpec((nhid, nclass), lambda t: (0, 0))],
        out_specs=[pl.BlockSpec((bm, nhid), lambda t: (t, 0)),
                   pl.BlockSpec((bm, nclass), lambda t: (t, 0)),
                   pl.BlockSpec((bm, nclass), lambda t: (t, 0))],
        out_shape=[jax.ShapeDtypeStruct((n, nhid), jnp.float32),
                   jax.ShapeDtypeStruct((n, nclass), jnp.float32),
                   jax.ShapeDtypeStruct((n, nclass), jnp.float32)],
        scratch_shapes=[pltpu.VMEM((n, ncat), jnp.float32)],
        compiler_params=pltpu.CompilerParams(
            dimension_semantics=("arbitrary",),
            vmem_limit_bytes=110 * 1024 * 1024),
    )(adj, x, W1, b1.reshape(1, nhid), W2)

    s2p = jnp.pad(s2, ((0, n_pad - n), (0, 0)))

    ts, ks = [], []
    for t in range(nblk):
        for k in range((t * bm) // bk, nkblk):
            ts.append(t)
            ks.append(k)
    idx = jnp.asarray([ts, ks], dtype=jnp.int32)
    nsteps = len(ts)

    def _pass_b_kernel(idx_ref, adj_ref, s2_ref, part_ref, out_ref, acc_ref):
        i = pl.program_id(0)
        t = idx_ref[0, i]
        k = idx_ref[1, i]

        @pl.when(i == 0)
        def _():
            acc_ref[...] = part_ref[...]

        lo = t * bm
        row = k * bk + jax.lax.broadcasted_iota(jnp.int32, (bk, 1), 0)
        s2_blk = jnp.where(row >= lo, s2_ref[pl.ds(k * bk, bk), :], 0.0)
        col = k * bk + jax.lax.broadcasted_iota(jnp.int32, (1, bk), 1)
        contrib = jax.lax.cond(
            k == nkblk - 1,
            lambda: jnp.dot(jnp.where(col < n, adj_ref[...], 0.0), s2_blk,
                            preferred_element_type=jnp.float32),
            lambda: jnp.dot(adj_ref[...], s2_blk,
                            preferred_element_type=jnp.float32))
        acc_ref[pl.ds(t * bm, bm), :] += contrib

        @pl.when(i == nsteps - 1)
        def _():
            out_ref[...] = acc_ref[...]

    out = pl.pallas_call(
        _pass_b_kernel,
        grid_spec=pltpu.PrefetchScalarGridSpec(
            num_scalar_prefetch=1,
            grid=(nsteps,),
            in_specs=[
                pl.BlockSpec((bm, bk),
                             lambda i, idx_ref: (idx_ref[0, i], idx_ref[1, i])),
                pl.BlockSpec((n_pad, nclass), lambda i, idx_ref: (0, 0)),
                pl.BlockSpec((n, nclass), lambda i, idx_ref: (0, 0)),
            ],
            out_specs=pl.BlockSpec((n, nclass), lambda i, idx_ref: (0, 0)),
            scratch_shapes=[pltpu.VMEM((n, nclass), jnp.float32)],
        ),
        out_shape=jax.ShapeDtypeStruct((n, nclass), jnp.float32),
        compiler_params=pltpu.CompilerParams(
            dimension_semantics=("arbitrary",),
            vmem_limit_bytes=100 * 1024 * 1024),
    )(idx, adj, s2p, part)

    return (hid, out)


# speedup vs baseline: 1.1598x; 1.1598x over previous
import jax
import jax.numpy as jnp
from jax.experimental import pallas as pl
from jax.experimental.pallas import tpu as pltpu

_BM = 400
_BK = 2048


def kernel(x, adj, W1, b1, W2):
    n, nfeat = x.shape
    nhid = W1.shape[1]
    nclass = W2.shape[1]
    bm = _BM
    bk = _BK
    nblk = n // bm
    nkblk = -(-n // bk)
    n_pad = nkblk * bk
    ncat = nhid + nclass

    def _pass_a_kernel(adj_ref, x_ref, w1_ref, b1_ref, w2_ref,
                       hid_ref, s2_ref, part_ref, cat_ref):
        t = pl.program_id(0)

        @pl.when(t == 0)
        def _():
            cat_ref[:, nhid:] = jnp.zeros((n, nclass), jnp.float32)
            cat_ref[:, :nhid] = jnp.dot(x_ref[...], w1_ref[...],
                                        preferred_element_type=jnp.float32)

        both = jnp.dot(adj_ref[...], cat_ref[...],
                       preferred_element_type=jnp.float32)
        h = jnp.maximum(both[:, :nhid] + b1_ref[...], 0.0)
        hid_ref[...] = h
        part_ref[...] = both[:, nhid:]
        s2_blk = jnp.dot(h, w2_ref[...], preferred_element_type=jnp.float32)
        cat_ref[pl.ds(t * bm, bm), nhid:] = s2_blk
        s2_ref[...] = s2_blk

    hid, s2, part = pl.pallas_call(
        _pass_a_kernel,
        grid=(nblk,),
        in_specs=[pl.BlockSpec((bm, n), lambda t: (t, 0)),
                  pl.BlockSpec((n, nfeat), lambda t: (0, 0)),
                  pl.BlockSpec((nfeat, nhid), lambda t: (0, 0)),
                  pl.BlockSpec((1, nhid), lambda t: (0, 0)),
                  pl.BlockSpec((nhid, nclass), lambda t: (0, 0))],
        out_specs=[pl.BlockSpec((bm, nhid), lambda t: (t, 0)),
                   pl.BlockSpec((bm, nclass), lambda t: (t, 0)),
                   pl.BlockSpec((bm, nclass), lambda t: (t, 0))],
        out_shape=[jax.ShapeDtypeStruct((n, nhid), jnp.float32),
                   jax.ShapeDtypeStruct((n, nclass), jnp.float32),
                   jax.ShapeDtypeStruct((n, nclass), jnp.float32)],
        scratch_shapes=[pltpu.VMEM((n, ncat), jnp.float32)],
        compiler_params=pltpu.CompilerParams(
            dimension_semantics=("arbitrary",),
            vmem_limit_bytes=110 * 1024 * 1024),
    )(adj, x, W1, b1.reshape(1, nhid), W2)

    s2p = jnp.pad(s2, ((0, n_pad - n), (0, 0)))

    ents = []
    for t in range(nblk):
        for k in range((t * bm) // bk, nkblk):
            ents.append((t, k, t * bm))
    if len(ents) % 2:
        ents.append((0, 0, n))
    e0 = ents[0::2]
    e1 = ents[1::2]
    nsteps = len(e0)
    idx = jnp.asarray(
        [[a[0] for a in e0], [a[1] for a in e0], [a[2] for a in e0],
         [b[0] for b in e1], [b[1] for b in e1], [b[2] for b in e1]],
        dtype=jnp.int32)

    def _pass_b_kernel(idx_ref, a0_ref, a1_ref, s2_ref, part_ref,
                       out_ref, acc_ref):
        i = pl.program_id(0)

        @pl.when(i == 0)
        def _():
            acc_ref[...] = part_ref[...]

        def one(t, k, lo, a_ref):
            col = k * bk + jax.lax.broadcasted_iota(jnp.int32, (1, bk), 1)
            a = jnp.where((col >= lo) & (col < n), a_ref[...], 0.0)
            acc_ref[pl.ds(t * bm, bm), :] += jnp.dot(
                a, s2_ref[pl.ds(k * bk, bk), :],
                preferred_element_type=jnp.float32)

        one(idx_ref[0, i], idx_ref[1, i], idx_ref[2, i], a0_ref)
        one(idx_ref[3, i], idx_ref[4, i], idx_ref[5, i], a1_ref)

        @pl.when(i == nsteps - 1)
        def _():
            out_ref[...] = acc_ref[...]

    out = pl.pallas_call(
        _pass_b_kernel,
        grid_spec=pltpu.PrefetchScalarGridSpec(
            num_scalar_prefetch=1,
            grid=(nsteps,),
            in_specs=[
                pl.BlockSpec((bm, bk),
                             lambda i, idx_ref: (idx_ref[0, i], idx_ref[1, i])),
                pl.BlockSpec((bm, bk),
                             lambda i, idx_ref: (idx_ref[3, i], idx_ref[4, i])),
                pl.BlockSpec((n_pad, nclass), lambda i, idx_ref: (0, 0)),
                pl.BlockSpec((n, nclass), lambda i, idx_ref: (0, 0)),
            ],
            out_specs=pl.BlockSpec((n, nclass), lambda i, idx_ref: (0, 0)),
            scratch_shapes=[pltpu.VMEM((n, nclass), jnp.float32)],
        ),
        out_shape=jax.ShapeDtypeStruct((n, nclass), jnp.float32),
        compiler_params=pltpu.CompilerParams(
            dimension_semantics=("arbitrary",),
            vmem_limit_bytes=100 * 1024 * 1024),
    )(idx, adj, adj, s2p, part)

    return (hid, out)


# speedup vs baseline: 1.1983x; 1.0332x over previous
import jax
import jax.numpy as jnp
from jax.experimental import pallas as pl
from jax.experimental.pallas import tpu as pltpu

_BM = 400
_BK = 2048
_NSTREAM = 4


def kernel(x, adj, W1, b1, W2):
    n, nfeat = x.shape
    nhid = W1.shape[1]
    nclass = W2.shape[1]
    bm = _BM
    bk = _BK
    hm = bm // 2
    nblk = n // bm
    nkblk = -(-n // bk)
    n_pad = nkblk * bk
    ncat = nhid + nclass

    def _pass_a_kernel(a0_ref, a1_ref, x_ref, w1_ref, b1_ref, w2_ref,
                       hid_ref, s2_ref, part_ref, cat_ref):
        t = pl.program_id(0)

        @pl.when(t == 0)
        def _():
            cat_ref[:, nhid:] = jnp.zeros((n, nclass), jnp.float32)
            cat_ref[:, :nhid] = jnp.dot(x_ref[...], w1_ref[...],
                                        preferred_element_type=jnp.float32)

        both0 = jnp.dot(a0_ref[...], cat_ref[...],
                        preferred_element_type=jnp.float32)
        both1 = jnp.dot(a1_ref[...], cat_ref[...],
                        preferred_element_type=jnp.float32)
        h0 = jnp.maximum(both0[:, :nhid] + b1_ref[...], 0.0)
        h1 = jnp.maximum(both1[:, :nhid] + b1_ref[...], 0.0)
        hid_ref[:hm, :] = h0
        hid_ref[hm:, :] = h1
        part_ref[:hm, :] = both0[:, nhid:]
        part_ref[hm:, :] = both1[:, nhid:]
        s0 = jnp.dot(h0, w2_ref[...], preferred_element_type=jnp.float32)
        s1v = jnp.dot(h1, w2_ref[...], preferred_element_type=jnp.float32)
        cat_ref[pl.ds(t * bm, hm), nhid:] = s0
        cat_ref[pl.ds(t * bm + hm, hm), nhid:] = s1v
        s2_ref[:hm, :] = s0
        s2_ref[hm:, :] = s1v

    hid, s2, part = pl.pallas_call(
        _pass_a_kernel,
        grid=(nblk,),
        in_specs=[pl.BlockSpec((hm, n), lambda t: (2 * t, 0)),
                  pl.BlockSpec((hm, n), lambda t: (2 * t + 1, 0)),
                  pl.BlockSpec((n, nfeat), lambda t: (0, 0)),
                  pl.BlockSpec((nfeat, nhid), lambda t: (0, 0)),
                  pl.BlockSpec((1, nhid), lambda t: (0, 0)),
                  pl.BlockSpec((nhid, nclass), lambda t: (0, 0))],
        out_specs=[pl.BlockSpec((bm, nhid), lambda t: (t, 0)),
                   pl.BlockSpec((bm, nclass), lambda t: (t, 0)),
                   pl.BlockSpec((bm, nclass), lambda t: (t, 0))],
        out_shape=[jax.ShapeDtypeStruct((n, nhid), jnp.float32),
                   jax.ShapeDtypeStruct((n, nclass), jnp.float32),
                   jax.ShapeDtypeStruct((n, nclass), jnp.float32)],
        scratch_shapes=[pltpu.VMEM((n, ncat), jnp.float32)],
        compiler_params=pltpu.CompilerParams(
            dimension_semantics=("arbitrary",),
            vmem_limit_bytes=110 * 1024 * 1024),
    )(adj, adj, x, W1, b1.reshape(1, nhid), W2)

    s2p = jnp.pad(s2, ((0, n_pad - n), (0, 0)))

    ents = []
    for t in range(nblk):
        for k in range((t * bm) // bk, nkblk):
            ents.append((t, k, t * bm))
    while len(ents) % _NSTREAM:
        ents.append((0, 0, n))
    lanes = [ents[s::_NSTREAM] for s in range(_NSTREAM)]
    nsteps = len(lanes[0])
    idx = jnp.asarray(
        [row for lane in lanes for row in
         ([a[0] for a in lane], [a[1] for a in lane], [a[2] for a in lane])],
        dtype=jnp.int32)

    def _pass_b_kernel(idx_ref, a0_ref, a1_ref, a2_ref, a3_ref,
                       s2_ref, part_ref, out_ref, acc_ref):
        i = pl.program_id(0)

        @pl.when(i == 0)
        def _():
            acc_ref[...] = part_ref[...]

        def one(s, a_ref):
            t = idx_ref[3 * s + 0, i]
            k = idx_ref[3 * s + 1, i]
            lo = idx_ref[3 * s + 2, i]
            col = k * bk + jax.lax.broadcasted_iota(jnp.int32, (1, bk), 1)
            a = jnp.where((col >= lo) & (col < n), a_ref[...], 0.0)
            acc_ref[pl.ds(t * bm, bm), :] += jnp.dot(
                a, s2_ref[pl.ds(k * bk, bk), :],
                preferred_element_type=jnp.float32)

        one(0, a0_ref)
        one(1, a1_ref)
        one(2, a2_ref)
        one(3, a3_ref)

        @pl.when(i == nsteps - 1)
        def _():
            out_ref[...] = acc_ref[...]

    adj_spec = [
        pl.BlockSpec(
            (bm, bk),
            (lambda s: lambda i, idx_ref:
             (idx_ref[3 * s, i], idx_ref[3 * s + 1, i]))(s))
        for s in range(_NSTREAM)
    ]

    out = pl.pallas_call(
        _pass_b_kernel,
        grid_spec=pltpu.PrefetchScalarGridSpec(
            num_scalar_prefetch=1,
            grid=(nsteps,),
            in_specs=adj_spec + [
                pl.BlockSpec((n_pad, nclass), lambda i, idx_ref: (0, 0)),
                pl.BlockSpec((n, nclass), lambda i, idx_ref: (0, 0)),
            ],
            out_specs=pl.BlockSpec((n, nclass), lambda i, idx_ref: (0, 0)),
            scratch_shapes=[pltpu.VMEM((n, nclass), jnp.float32)],
        ),
        out_shape=jax.ShapeDtypeStruct((n, nclass), jnp.float32),
        compiler_params=pltpu.CompilerParams(
            dimension_semantics=("arbitrary",),
            vmem_limit_bytes=100 * 1024 * 1024),
    )(idx, adj, adj, adj, adj, s2p, part)

    return (hid, out)
